# baseline (speedup 1.0000x reference)
import functools

import jax
import jax.numpy as jnp
from jax import lax
from jax.experimental import pallas as pl
from jax.experimental.pallas import tpu as pltpu
from jax.experimental.pallas import tpu_sc as plsc

NC = 2
NS = 16
NW = NC * NS
LANES = 16

INV_SQRT2 = 0.70710678118654752

O, T, D = 10000, 320000, 128
N = O + T
R = 2000
CH = 80
NCH = N // CH
HEAD_CH = O // CH
MID_END_CH = T // CH
CPW = (NCH + NW - 1) // NW
IDXPAD = NW * CPW
IH1 = 64
IBUF = 72
CPW_PAD = IH1 + IBUF
SUB = 100
EPW = T // NW
QT = 624
PIECE = 104
REM = O - NS * QT
HW = 16


def _hist_body(o_hbm, cpart_hbm, oidx, ones_buf, zbuf, hist_sh):
    cid = lax.axis_index("c")
    sid = lax.axis_index("s")
    wid = sid * NC + cid

    zero16 = jnp.zeros((LANES,), jnp.float32)
    ones16 = jnp.ones((LANES,), jnp.float32)

    def zl(r, carry):
        zbuf[r, pl.ds(0, HW)] = zero16
        return carry

    lax.fori_loop(0, PIECE, zl, 0)

    def ol(r, carry):
        ones_buf[r, pl.ds(0, HW)] = ones16
        return carry

    lax.fori_loop(0, CH, ol, 0)

    for q in range(QT // PIECE):
        pltpu.sync_copy(zbuf, hist_sh.at[pl.ds(sid * QT + q * PIECE, PIECE)])

    @pl.when(sid == 0)
    def _():
        pltpu.sync_copy(zbuf.at[pl.ds(0, REM)], hist_sh.at[pl.ds(NS * QT, REM)])

    plsc.subcore_barrier()

    def hloop(r, carry):
        pltpu.sync_copy(o_hbm.at[pl.ds(wid * EPW + r * CH, CH)], oidx)
        pltpu.sync_copy(ones_buf, hist_sh.at[oidx], add=True)
        return carry

    lax.fori_loop(0, EPW // CH, hloop, 0)
    plsc.subcore_barrier()

    for q in range(QT // PIECE):
        sl = pl.ds(sid * QT + q * PIECE, PIECE)
        pltpu.sync_copy(hist_sh.at[sl], cpart_hbm.at[cid, sl])

    @pl.when(sid == 0)
    def _():
        sl = pl.ds(NS * QT, REM)
        pltpu.sync_copy(hist_sh.at[sl], cpart_hbm.at[cid, sl])


_hist = pl.kernel(
    _hist_body,
    out_type=jax.ShapeDtypeStruct((NC, O, HW), jnp.float32),
    mesh=plsc.VectorSubcoreMesh(core_axis_name="c", subcore_axis_name="s"),
    scratch_types=[
        pltpu.VMEM((CH,), jnp.int32),
        pltpu.VMEM((CH, HW), jnp.float32),
        pltpu.VMEM((PIECE, HW), jnp.float32),
        pltpu.VMEM_SHARED((O, HW), jnp.float32),
    ],
)


def _mm_obj_body(x_ref, wt_ref, cpart_ref, g_ref):
    h = jnp.dot(x_ref[...], wt_ref[...], preferred_element_type=jnp.float32)
    c = jnp.sum(cpart_ref[...], axis=(0, 1, 3)) * jnp.float32(1.0 / HW)
    dinv = lax.rsqrt(c + 2.0)
    g_ref[...] = h * dinv[:, None]


def _mm_pred_body(x_ref, wt_ref, g_ref):
    i = pl.program_id(0)
    h = jnp.dot(x_ref[...], wt_ref[...], preferred_element_type=jnp.float32)
    scale = jnp.where(i < (T - O) // R, jnp.float32(INV_SQRT2), jnp.float32(1.0))
    g_ref[...] = h * scale


def _mm_obj(x, wt, cpart):
    return pl.pallas_call(
        _mm_obj_body,
        grid=(O // R,),
        in_specs=[
            pl.BlockSpec((R, D), lambda i: (i, 0)),
            pl.BlockSpec((D, D), lambda i: (0, 0)),
            pl.BlockSpec((NC, 1, R, HW), lambda i: (0, i, 0, 0)),
        ],
        out_specs=pl.BlockSpec((R, D), lambda i: (i, 0)),
        out_shape=jax.ShapeDtypeStruct((O, D), jnp.float32),
    )(x, wt, cpart)


def _mm_pred(x, wt):
    return pl.pallas_call(
        _mm_pred_body,
        grid=(T // R,),
        in_specs=[
            pl.BlockSpec((R, D), lambda i: (i, 0)),
            pl.BlockSpec((D, D), lambda i: (0, 0)),
        ],
        out_specs=pl.BlockSpec((R, D), lambda i: (i, 0)),
        out_shape=jax.ShapeDtypeStruct((T, D), jnp.float32),
    )(x, wt)


def _main_body(gobj_hbm, gpred_hbm, s_hbm, o_hbm, b_hbm,
               outp_hbm, yhead_hbm, accp_hbm,
               g_loc, gath, sidx, oidx, b_loc, zrow, acc_sh,
               sem_g, sem_i, sem_p):
    cid = lax.axis_index("c")
    sid = lax.axis_index("s")
    wid = sid * NC + cid

    zero16 = jnp.zeros((LANES,), jnp.float32)

    def zloop(r, carry):
        for j in range(D // LANES):
            zrow[r, pl.ds(j * LANES, LANES)] = zero16
        return carry

    lax.fori_loop(0, PIECE, zloop, 0)
    for q in range(QT // PIECE):
        pltpu.sync_copy(zrow, acc_sh.at[pl.ds(sid * QT + q * PIECE, PIECE)])

    @pl.when(sid == 0)
    def _():
        pltpu.sync_copy(zrow.at[pl.ds(0, REM)], acc_sh.at[pl.ds(NS * QT, REM)])

    pltpu.sync_copy(b_hbm, b_loc)
    plsc.subcore_barrier()

    inv2 = jnp.float32(INV_SQRT2)

    def issue_g(ch2, buf2):
        @pl.when(ch2 < NCH)
        def _():
            b2 = ch2 * CH

            @pl.when(ch2 < HEAD_CH)
            def _():
                pltpu.async_copy(gobj_hbm.at[pl.ds(b2, CH)],
                                 g_loc.at[buf2], sem_p)

            @pl.when(ch2 >= HEAD_CH)
            def _():
                pltpu.async_copy(gpred_hbm.at[pl.ds(b2 - O, CH)],
                                 g_loc.at[buf2], sem_p)

    issue_g(wid, 0)

    def chunk(i, carry):
        ch = i * NW + wid

        @pl.when(ch < NCH)
        def _():
            base = ch * CH
            buf = lax.rem(i, 2)
            is_head = ch < HEAD_CH
            is_tail = ch >= MID_END_CH
            is_mid = jnp.logical_and(jnp.logical_not(is_head),
                                     jnp.logical_not(is_tail))

            @pl.when(jnp.logical_not(is_tail))
            def _():
                d_s = pltpu.async_copy(s_hbm.at[pl.ds(base, CH)], sidx, sem_i)
                d_o = pltpu.async_copy(o_hbm.at[pl.ds(base, CH)], oidx, sem_i)
                d_s.wait()
                d_o.wait()

            pltpu.make_async_copy(
                gpred_hbm.at[pl.ds(0, CH)], g_loc.at[buf], sem_p).wait()
            issue_g(ch + NW, 1 - buf)

            @pl.when(jnp.logical_not(is_tail))
            def _():
                pltpu.async_copy(gobj_hbm.at[sidx], gath, sem_g).wait()
                pltpu.sync_copy(g_loc.at[buf], acc_sh.at[oidx], add=True)

            @pl.when(is_head)
            def _():
                def rl(r, c2):
                    for j in range(D // LANES):
                        sl = pl.ds(j * LANES, LANES)
                        gath[r, sl] = gath[r, sl] + g_loc[buf, r, sl]
                    return c2

                lax.fori_loop(0, CH, rl, 0)
                pltpu.sync_copy(gath, yhead_hbm.at[pl.ds(base, CH)])

            @pl.when(is_mid)
            def _():
                def rl(r, c2):
                    for j in range(D // LANES):
                        sl = pl.ds(j * LANES, LANES)
                        gath[r, sl] = ((gath[r, sl] + g_loc[buf, r, sl])
                                       * inv2 + b_loc[sl])
                    return c2

                lax.fori_loop(0, CH, rl, 0)
                pltpu.sync_copy(gath, outp_hbm.at[pl.ds(base - O, CH)])

            @pl.when(is_tail)
            def _():
                def rl(r, c2):
                    for j in range(D // LANES):
                        sl = pl.ds(j * LANES, LANES)
                        gath[r, sl] = g_loc[buf, r, sl] + b_loc[sl]
                    return c2

                lax.fori_loop(0, CH, rl, 0)
                pltpu.sync_copy(gath, outp_hbm.at[pl.ds(base - O, CH)])

        return carry

    lax.fori_loop(0, CPW, chunk, 0)
    plsc.subcore_barrier()

    for q in range(QT // PIECE):
        sl = pl.ds(sid * QT + q * PIECE, PIECE)
        pltpu.sync_copy(acc_sh.at[sl], accp_hbm.at[cid, sl])

    @pl.when(sid == 0)
    def _():
        sl = pl.ds(NS * QT, REM)
        pltpu.sync_copy(acc_sh.at[sl], accp_hbm.at[cid, sl])


_main_sc = pl.kernel(
    _main_body,
    out_type=(
        jax.ShapeDtypeStruct((T, D), jnp.float32),
        jax.ShapeDtypeStruct((O, D), jnp.float32),
        jax.ShapeDtypeStruct((NC, O, D), jnp.float32),
    ),
    mesh=plsc.VectorSubcoreMesh(core_axis_name="c", subcore_axis_name="s"),
    scratch_types=[
        pltpu.VMEM((2, CH, D), jnp.float32),
        pltpu.VMEM((CH, D), jnp.float32),
        pltpu.VMEM((CH,), jnp.int32),
        pltpu.VMEM((CH,), jnp.int32),
        pltpu.VMEM((D,), jnp.float32),
        pltpu.VMEM((PIECE, D), jnp.float32),
        pltpu.VMEM_SHARED((O, D), jnp.float32),
        pltpu.SemaphoreType.DMA,
        pltpu.SemaphoreType.DMA,
        pltpu.SemaphoreType.DMA,
    ],
)


def _final_obj_body(y_ref, acc_ref, cpart_ref, b_ref, out_ref):
    a = acc_ref[...]
    accsum = a[0] + a[1]
    c = jnp.sum(cpart_ref[...], axis=(0, 1, 3)) * jnp.float32(1.0 / HW)
    dinv = lax.rsqrt(c + 2.0)
    out_ref[...] = dinv[:, None] * (y_ref[...] + accsum) + b_ref[...]


def _final_obj(yhead, accp, cpart, b2):
    return pl.pallas_call(
        _final_obj_body,
        grid=(O // R,),
        in_specs=[
            pl.BlockSpec((R, D), lambda i: (i, 0)),
            pl.BlockSpec((NC, R, D), lambda i: (0, i, 0)),
            pl.BlockSpec((NC, 1, R, HW), lambda i: (0, i, 0, 0)),
            pl.BlockSpec((1, D), lambda i: (0, 0)),
        ],
        out_specs=pl.BlockSpec((R, D), lambda i: (i, 0)),
        out_shape=jax.ShapeDtypeStruct((O, D), jnp.float32),
    )(yhead, accp, cpart, b2)


def kernel(obj_vecs, pred_vecs, edges, W, b):
    s = edges[:, 0]
    o = edges[:, 1]
    wt = W.T
    cpart = _hist(o)
    cpart4 = cpart.reshape(NC, O // R, R, HW)
    g_obj = _mm_obj(obj_vecs, wt, cpart4)
    g_pred = _mm_pred(pred_vecs, wt)
    out_pred, yhead, accp = _main_sc(g_obj, g_pred, s, o, b)
    out_obj = _final_obj(yhead, accp, cpart4, b.reshape(1, D))
    return (out_obj, out_pred)


# speedup vs baseline: 1.0752x; 1.0752x over previous
import functools

import jax
import jax.numpy as jnp
from jax import lax
from jax.experimental import pallas as pl
from jax.experimental.pallas import tpu as pltpu
from jax.experimental.pallas import tpu_sc as plsc

NC = 2
NS = 16
NW = NC * NS
LANES = 16

INV_SQRT2 = 0.70710678118654752

O, T, D = 10000, 320000, 128
N = O + T
R = 2000
CH = 80
NCH = N // CH
HEAD_CH = O // CH
MID_END_CH = T // CH
CPW = (NCH + NW - 1) // NW
IDXPAD = NW * CPW
IH1 = 64
IBUF = 72
CPW_PAD = IH1 + IBUF
SUB = 100
EPW = T // NW
QT = 624
PIECE = 208
REM = O - NS * QT
HW = 16


def _hist_body(o_hbm, cpart_hbm, oidx, ones_buf, zbuf, hist_sh):
    cid = lax.axis_index("c")
    sid = lax.axis_index("s")
    wid = sid * NC + cid

    zero16 = jnp.zeros((LANES,), jnp.float32)
    ones16 = jnp.ones((LANES,), jnp.float32)

    def zl(r, carry):
        zbuf[r, pl.ds(0, HW)] = zero16
        return carry

    lax.fori_loop(0, PIECE, zl, 0)

    def ol(r, carry):
        ones_buf[r, pl.ds(0, HW)] = ones16
        return carry

    lax.fori_loop(0, CH, ol, 0)

    for q in range(QT // PIECE):
        pltpu.sync_copy(zbuf, hist_sh.at[pl.ds(sid * QT + q * PIECE, PIECE)])

    @pl.when(sid == 0)
    def _():
        pltpu.sync_copy(zbuf.at[pl.ds(0, REM)], hist_sh.at[pl.ds(NS * QT, REM)])

    plsc.subcore_barrier()

    def hloop(r, carry):
        pltpu.sync_copy(o_hbm.at[pl.ds(wid * EPW + r * CH, CH)], oidx)
        pltpu.sync_copy(ones_buf, hist_sh.at[oidx], add=True)
        return carry

    lax.fori_loop(0, EPW // CH, hloop, 0)
    plsc.subcore_barrier()

    for q in range(QT // PIECE):
        sl = pl.ds(sid * QT + q * PIECE, PIECE)
        pltpu.sync_copy(hist_sh.at[sl], cpart_hbm.at[cid, sl])

    @pl.when(sid == 0)
    def _():
        sl = pl.ds(NS * QT, REM)
        pltpu.sync_copy(hist_sh.at[sl], cpart_hbm.at[cid, sl])


_hist = pl.kernel(
    _hist_body,
    out_type=jax.ShapeDtypeStruct((NC, O, HW), jnp.float32),
    mesh=plsc.VectorSubcoreMesh(core_axis_name="c", subcore_axis_name="s"),
    scratch_types=[
        pltpu.VMEM((CH,), jnp.int32),
        pltpu.VMEM((CH, HW), jnp.float32),
        pltpu.VMEM((PIECE, HW), jnp.float32),
        pltpu.VMEM_SHARED((O, HW), jnp.float32),
    ],
)


def _mm_obj_body(x_ref, wt_ref, cpart_ref, g_ref):
    h = jnp.dot(x_ref[...], wt_ref[...], preferred_element_type=jnp.float32)
    c = jnp.sum(cpart_ref[...], axis=(0, 1, 3)) * jnp.float32(1.0 / HW)
    dinv = lax.rsqrt(c + 2.0)
    g_ref[...] = h * dinv[:, None]


def _mm_pred_body(x_ref, wt_ref, g_ref):
    i = pl.program_id(0)
    h = jnp.dot(x_ref[...], wt_ref[...], preferred_element_type=jnp.float32)
    scale = jnp.where(i < (T - O) // R, jnp.float32(INV_SQRT2), jnp.float32(1.0))
    g_ref[...] = h * scale


def _mm_obj(x, wt, cpart):
    return pl.pallas_call(
        _mm_obj_body,
        grid=(O // R,),
        in_specs=[
            pl.BlockSpec((R, D), lambda i: (i, 0)),
            pl.BlockSpec((D, D), lambda i: (0, 0)),
            pl.BlockSpec((NC, 1, R, HW), lambda i: (0, i, 0, 0)),
        ],
        out_specs=pl.BlockSpec((R, D), lambda i: (i, 0)),
        out_shape=jax.ShapeDtypeStruct((O, D), jnp.float32),
    )(x, wt, cpart)


def _mm_pred(x, wt):
    return pl.pallas_call(
        _mm_pred_body,
        grid=(T // R,),
        in_specs=[
            pl.BlockSpec((R, D), lambda i: (i, 0)),
            pl.BlockSpec((D, D), lambda i: (0, 0)),
        ],
        out_specs=pl.BlockSpec((R, D), lambda i: (i, 0)),
        out_shape=jax.ShapeDtypeStruct((T, D), jnp.float32),
    )(x, wt)


def _main_body(gobj_hbm, gpred_hbm, s_hbm, o_hbm, b_hbm,
               outp_hbm, yhead_hbm, accp_hbm,
               g_loc, gath, sidx, oidx, b_loc, zrow, acc_sh,
               sem_g, sem_st):
    cid = lax.axis_index("c")
    sid = lax.axis_index("s")
    wid = sid * NC + cid

    zero16 = jnp.zeros((LANES,), jnp.float32)

    def zloop(r, carry):
        for j in range(D // LANES):
            zrow[r, pl.ds(j * LANES, LANES)] = zero16
        return carry

    lax.fori_loop(0, PIECE, zloop, 0)
    for q in range(QT // PIECE):
        pltpu.sync_copy(zrow, acc_sh.at[pl.ds(sid * QT + q * PIECE, PIECE)])

    @pl.when(sid == 0)
    def _():
        pltpu.sync_copy(zrow.at[pl.ds(0, REM)], acc_sh.at[pl.ds(NS * QT, REM)])

    pltpu.sync_copy(b_hbm, b_loc)
    plsc.subcore_barrier()

    inv2 = jnp.float32(INV_SQRT2)

    def chunk(i, carry):
        ch = i * NW + wid

        @pl.when(ch < NCH)
        def _():
            base = ch * CH
            is_head = ch < HEAD_CH
            is_tail = ch >= MID_END_CH
            is_mid = jnp.logical_and(jnp.logical_not(is_head),
                                     jnp.logical_not(is_tail))

            @pl.when(i > 0)
            def _():
                pltpu.make_async_copy(
                    gath, outp_hbm.at[pl.ds(0, CH)], sem_st).wait()

            @pl.when(is_head)
            def _():
                pltpu.sync_copy(gobj_hbm.at[pl.ds(base, CH)], g_loc)

            @pl.when(jnp.logical_not(is_head))
            def _():
                pltpu.sync_copy(gpred_hbm.at[pl.ds(base - O, CH)], g_loc)

            @pl.when(jnp.logical_not(is_tail))
            def _():
                pltpu.sync_copy(s_hbm.at[pl.ds(base, CH)], sidx)
                pltpu.sync_copy(o_hbm.at[pl.ds(base, CH)], oidx)
                pltpu.async_copy(gobj_hbm.at[sidx], gath, sem_g).wait()
                pltpu.sync_copy(g_loc, acc_sh.at[oidx], add=True)

            @pl.when(is_head)
            def _():
                def rl(r, c2):
                    for j in range(D // LANES):
                        sl = pl.ds(j * LANES, LANES)
                        gath[r, sl] = gath[r, sl] + g_loc[r, sl]
                    return c2

                lax.fori_loop(0, CH, rl, 0)
                pltpu.async_copy(gath, yhead_hbm.at[pl.ds(base, CH)], sem_st)

            @pl.when(is_mid)
            def _():
                def rl(r, c2):
                    for j in range(D // LANES):
                        sl = pl.ds(j * LANES, LANES)
                        gath[r, sl] = ((gath[r, sl] + g_loc[r, sl]) * inv2
                                       + b_loc[sl])
                    return c2

                lax.fori_loop(0, CH, rl, 0)
                pltpu.async_copy(gath, outp_hbm.at[pl.ds(base - O, CH)],
                                 sem_st)

            @pl.when(is_tail)
            def _():
                def rl(r, c2):
                    for j in range(D // LANES):
                        sl = pl.ds(j * LANES, LANES)
                        gath[r, sl] = g_loc[r, sl] + b_loc[sl]
                    return c2

                lax.fori_loop(0, CH, rl, 0)
                pltpu.async_copy(gath, outp_hbm.at[pl.ds(base - O, CH)],
                                 sem_st)

        return carry

    lax.fori_loop(0, CPW, chunk, 0)
    plsc.subcore_barrier()

    for q in range(QT // PIECE):
        sl = pl.ds(sid * QT + q * PIECE, PIECE)
        pltpu.sync_copy(acc_sh.at[sl], accp_hbm.at[cid, sl])

    @pl.when(sid == 0)
    def _():
        sl = pl.ds(NS * QT, REM)
        pltpu.sync_copy(acc_sh.at[sl], accp_hbm.at[cid, sl])


_main_sc = pl.kernel(
    _main_body,
    out_type=(
        jax.ShapeDtypeStruct((T, D), jnp.float32),
        jax.ShapeDtypeStruct((O, D), jnp.float32),
        jax.ShapeDtypeStruct((NC, O, D), jnp.float32),
    ),
    mesh=plsc.VectorSubcoreMesh(core_axis_name="c", subcore_axis_name="s"),
    scratch_types=[
        pltpu.VMEM((CH, D), jnp.float32),
        pltpu.VMEM((CH, D), jnp.float32),
        pltpu.VMEM((CH,), jnp.int32),
        pltpu.VMEM((CH,), jnp.int32),
        pltpu.VMEM((D,), jnp.float32),
        pltpu.VMEM((PIECE, D), jnp.float32),
        pltpu.VMEM_SHARED((O, D), jnp.float32),
        pltpu.SemaphoreType.DMA,
        pltpu.SemaphoreType.DMA,
    ],
)


def _final_obj_body(y_ref, acc_ref, cpart_ref, b_ref, out_ref):
    a = acc_ref[...]
    accsum = a[0] + a[1]
    c = jnp.sum(cpart_ref[...], axis=(0, 1, 3)) * jnp.float32(1.0 / HW)
    dinv = lax.rsqrt(c + 2.0)
    out_ref[...] = dinv[:, None] * (y_ref[...] + accsum) + b_ref[...]


def _final_obj(yhead, accp, cpart, b2):
    return pl.pallas_call(
        _final_obj_body,
        grid=(O // R,),
        in_specs=[
            pl.BlockSpec((R, D), lambda i: (i, 0)),
            pl.BlockSpec((NC, R, D), lambda i: (0, i, 0)),
            pl.BlockSpec((NC, 1, R, HW), lambda i: (0, i, 0, 0)),
            pl.BlockSpec((1, D), lambda i: (0, 0)),
        ],
        out_specs=pl.BlockSpec((R, D), lambda i: (i, 0)),
        out_shape=jax.ShapeDtypeStruct((O, D), jnp.float32),
    )(yhead, accp, cpart, b2)


def kernel(obj_vecs, pred_vecs, edges, W, b):
    s = edges[:, 0]
    o = edges[:, 1]
    wt = W.T
    cpart = _hist(o)
    cpart4 = cpart.reshape(NC, O // R, R, HW)
    g_obj = _mm_obj(obj_vecs, wt, cpart4)
    g_pred = _mm_pred(pred_vecs, wt)
    out_pred, yhead, accp = _main_sc(g_obj, g_pred, s, o, b)
    out_obj = _final_obj(yhead, accp, cpart4, b.reshape(1, D))
    return (out_obj, out_pred)


# speedup vs baseline: 1.1216x; 1.0431x over previous
import functools

import jax
import jax.numpy as jnp
from jax import lax
from jax.experimental import pallas as pl
from jax.experimental.pallas import tpu as pltpu
from jax.experimental.pallas import tpu_sc as plsc

NC = 2
NS = 16
NW = NC * NS
LANES = 16

INV_SQRT2 = 0.70710678118654752

O, T, D = 10000, 320000, 128
N = O + T
R = 2000
CH = 80
NCH = N // CH
HEAD_CH = O // CH
MID_END_CH = T // CH
CPW = (NCH + NW - 1) // NW
IDXPAD = NW * CPW
IH1 = 64
IBUF = 72
CPW_PAD = IH1 + IBUF
SUB = 100
EPW = T // NW
QT = 624
PIECE = 208
REM = O - NS * QT
HW = 16


def _hist_body(o_hbm, cpart_hbm, oidx, ones_buf, zbuf, hist_sh):
    cid = lax.axis_index("c")
    sid = lax.axis_index("s")
    wid = sid * NC + cid

    zero16 = jnp.zeros((LANES,), jnp.float32)
    ones16 = jnp.ones((LANES,), jnp.float32)

    def zl(r, carry):
        zbuf[r, pl.ds(0, HW)] = zero16
        return carry

    lax.fori_loop(0, PIECE, zl, 0)

    def ol(r, carry):
        ones_buf[r, pl.ds(0, HW)] = ones16
        return carry

    lax.fori_loop(0, CH, ol, 0)

    for q in range(QT // PIECE):
        pltpu.sync_copy(zbuf, hist_sh.at[pl.ds(sid * QT + q * PIECE, PIECE)])

    @pl.when(sid == 0)
    def _():
        pltpu.sync_copy(zbuf.at[pl.ds(0, REM)], hist_sh.at[pl.ds(NS * QT, REM)])

    plsc.subcore_barrier()

    def hloop(r, carry):
        pltpu.sync_copy(o_hbm.at[pl.ds(wid * EPW + r * CH, CH)], oidx)
        pltpu.sync_copy(ones_buf, hist_sh.at[oidx], add=True)
        return carry

    lax.fori_loop(0, EPW // CH, hloop, 0)
    plsc.subcore_barrier()

    for q in range(QT // PIECE):
        sl = pl.ds(sid * QT + q * PIECE, PIECE)
        pltpu.sync_copy(hist_sh.at[sl], cpart_hbm.at[cid, sl])

    @pl.when(sid == 0)
    def _():
        sl = pl.ds(NS * QT, REM)
        pltpu.sync_copy(hist_sh.at[sl], cpart_hbm.at[cid, sl])


_hist = pl.kernel(
    _hist_body,
    out_type=jax.ShapeDtypeStruct((NC, O, HW), jnp.float32),
    mesh=plsc.VectorSubcoreMesh(core_axis_name="c", subcore_axis_name="s"),
    scratch_types=[
        pltpu.VMEM((CH,), jnp.int32),
        pltpu.VMEM((CH, HW), jnp.float32),
        pltpu.VMEM((PIECE, HW), jnp.float32),
        pltpu.VMEM_SHARED((O, HW), jnp.float32),
    ],
)


def _mm_obj_body(x_ref, wt_ref, cpart_ref, g_ref):
    h = jnp.dot(x_ref[...], wt_ref[...], preferred_element_type=jnp.float32)
    c = jnp.sum(cpart_ref[...], axis=(0, 1, 3)) * jnp.float32(1.0 / HW)
    dinv = lax.rsqrt(c + 2.0)
    g_ref[...] = h * dinv[:, None]


def _mm_pred_body(x_ref, wt_ref, g_ref):
    i = pl.program_id(0)
    h = jnp.dot(x_ref[...], wt_ref[...], preferred_element_type=jnp.float32)
    scale = jnp.where(i < (T - O) // R, jnp.float32(INV_SQRT2), jnp.float32(1.0))
    g_ref[...] = h * scale


def _mm_obj(x, wt, cpart):
    return pl.pallas_call(
        _mm_obj_body,
        grid=(O // R,),
        in_specs=[
            pl.BlockSpec((R, D), lambda i: (i, 0)),
            pl.BlockSpec((D, D), lambda i: (0, 0)),
            pl.BlockSpec((NC, 1, R, HW), lambda i: (0, i, 0, 0)),
        ],
        out_specs=pl.BlockSpec((R, D), lambda i: (i, 0)),
        out_shape=jax.ShapeDtypeStruct((O, D), jnp.float32),
    )(x, wt, cpart)


def _mm_pred(x, wt):
    return pl.pallas_call(
        _mm_pred_body,
        grid=(T // R,),
        in_specs=[
            pl.BlockSpec((R, D), lambda i: (i, 0)),
            pl.BlockSpec((D, D), lambda i: (0, 0)),
        ],
        out_specs=pl.BlockSpec((R, D), lambda i: (i, 0)),
        out_shape=jax.ShapeDtypeStruct((T, D), jnp.float32),
    )(x, wt)


def _main_body(gobj_hbm, gpred_hbm, s_hbm, o_hbm, b_hbm,
               outp_hbm, yhead_hbm, accp_hbm,
               g_loc, gath, sidx, oidx, b_loc, zrow, acc_sh,
               sem_g, sem_i, sem_st):
    cid = lax.axis_index("c")
    sid = lax.axis_index("s")
    wid = sid * NC + cid

    zero16 = jnp.zeros((LANES,), jnp.float32)

    def zloop(r, carry):
        for j in range(D // LANES):
            zrow[r, pl.ds(j * LANES, LANES)] = zero16
        return carry

    lax.fori_loop(0, PIECE, zloop, 0)
    for q in range(QT // PIECE):
        pltpu.sync_copy(zrow, acc_sh.at[pl.ds(sid * QT + q * PIECE, PIECE)])

    @pl.when(sid == 0)
    def _():
        pltpu.sync_copy(zrow.at[pl.ds(0, REM)], acc_sh.at[pl.ds(NS * QT, REM)])

    pltpu.sync_copy(b_hbm, b_loc)
    plsc.subcore_barrier()

    inv2 = jnp.float32(INV_SQRT2)

    def chunk(i, carry):
        ch = i * NW + wid

        @pl.when(ch < NCH)
        def _():
            base = ch * CH
            is_head = ch < HEAD_CH
            is_tail = ch >= MID_END_CH
            is_mid = jnp.logical_and(jnp.logical_not(is_head),
                                     jnp.logical_not(is_tail))

            @pl.when(i > 0)
            def _():
                pltpu.make_async_copy(
                    gath, outp_hbm.at[pl.ds(0, CH)], sem_st).wait()

            @pl.when(is_head)
            def _():
                pltpu.sync_copy(gobj_hbm.at[pl.ds(base, CH)], g_loc)

            @pl.when(jnp.logical_not(is_head))
            def _():
                pltpu.sync_copy(gpred_hbm.at[pl.ds(base - O, CH)], g_loc)

            @pl.when(jnp.logical_not(is_tail))
            def _():
                d_s = pltpu.async_copy(s_hbm.at[pl.ds(base, CH)], sidx, sem_i)
                d_o = pltpu.async_copy(o_hbm.at[pl.ds(base, CH)], oidx, sem_i)
                d_s.wait()
                d_o.wait()
                pltpu.async_copy(gobj_hbm.at[sidx], gath, sem_g).wait()
                pltpu.sync_copy(g_loc, acc_sh.at[oidx], add=True)

            @pl.when(is_head)
            def _():
                def rl(r, c2):
                    for j in range(D // LANES):
                        sl = pl.ds(j * LANES, LANES)
                        gath[r, sl] = gath[r, sl] + g_loc[r, sl]
                    return c2

                lax.fori_loop(0, CH, rl, 0)
                pltpu.async_copy(gath, yhead_hbm.at[pl.ds(base, CH)], sem_st)

            @pl.when(is_mid)
            def _():
                def rl(r, c2):
                    for j in range(D // LANES):
                        sl = pl.ds(j * LANES, LANES)
                        gath[r, sl] = ((gath[r, sl] + g_loc[r, sl]) * inv2
                                       + b_loc[sl])
                    return c2

                lax.fori_loop(0, CH, rl, 0)
                pltpu.async_copy(gath, outp_hbm.at[pl.ds(base - O, CH)],
                                 sem_st)

            @pl.when(is_tail)
            def _():
                def rl(r, c2):
                    for j in range(D // LANES):
                        sl = pl.ds(j * LANES, LANES)
                        gath[r, sl] = g_loc[r, sl] + b_loc[sl]
                    return c2

                lax.fori_loop(0, CH, rl, 0)
                pltpu.async_copy(gath, outp_hbm.at[pl.ds(base - O, CH)],
                                 sem_st)

        return carry

    lax.fori_loop(0, CPW, chunk, 0)
    plsc.subcore_barrier()

    for q in range(QT // PIECE):
        sl = pl.ds(sid * QT + q * PIECE, PIECE)
        pltpu.sync_copy(acc_sh.at[sl], accp_hbm.at[cid, sl])

    @pl.when(sid == 0)
    def _():
        sl = pl.ds(NS * QT, REM)
        pltpu.sync_copy(acc_sh.at[sl], accp_hbm.at[cid, sl])


_main_sc = pl.kernel(
    _main_body,
    out_type=(
        jax.ShapeDtypeStruct((T, D), jnp.float32),
        jax.ShapeDtypeStruct((O, D), jnp.float32),
        jax.ShapeDtypeStruct((NC, O, D), jnp.float32),
    ),
    mesh=plsc.VectorSubcoreMesh(core_axis_name="c", subcore_axis_name="s"),
    scratch_types=[
        pltpu.VMEM((CH, D), jnp.float32),
        pltpu.VMEM((CH, D), jnp.float32),
        pltpu.VMEM((CH,), jnp.int32),
        pltpu.VMEM((CH,), jnp.int32),
        pltpu.VMEM((D,), jnp.float32),
        pltpu.VMEM((PIECE, D), jnp.float32),
        pltpu.VMEM_SHARED((O, D), jnp.float32),
        pltpu.SemaphoreType.DMA,
        pltpu.SemaphoreType.DMA,
        pltpu.SemaphoreType.DMA,
    ],
)


def _final_obj_body(y_ref, acc_ref, cpart_ref, b_ref, out_ref):
    a = acc_ref[...]
    accsum = a[0] + a[1]
    c = jnp.sum(cpart_ref[...], axis=(0, 1, 3)) * jnp.float32(1.0 / HW)
    dinv = lax.rsqrt(c + 2.0)
    out_ref[...] = dinv[:, None] * (y_ref[...] + accsum) + b_ref[...]


def _final_obj(yhead, accp, cpart, b2):
    return pl.pallas_call(
        _final_obj_body,
        grid=(O // R,),
        in_specs=[
            pl.BlockSpec((R, D), lambda i: (i, 0)),
            pl.BlockSpec((NC, R, D), lambda i: (0, i, 0)),
            pl.BlockSpec((NC, 1, R, HW), lambda i: (0, i, 0, 0)),
            pl.BlockSpec((1, D), lambda i: (0, 0)),
        ],
        out_specs=pl.BlockSpec((R, D), lambda i: (i, 0)),
        out_shape=jax.ShapeDtypeStruct((O, D), jnp.float32),
    )(yhead, accp, cpart, b2)


def kernel(obj_vecs, pred_vecs, edges, W, b):
    s = edges[:, 0]
    o = edges[:, 1]
    wt = W.T
    cpart = _hist(o)
    cpart4 = cpart.reshape(NC, O // R, R, HW)
    g_obj = _mm_obj(obj_vecs, wt, cpart4)
    g_pred = _mm_pred(pred_vecs, wt)
    out_pred, yhead, accp = _main_sc(g_obj, g_pred, s, o, b)
    out_obj = _final_obj(yhead, accp, cpart4, b.reshape(1, D))
    return (out_obj, out_pred)
